# baseline (speedup 1.0000x reference)
import functools

import jax
import jax.numpy as jnp
from jax import lax
from jax.experimental import pallas as pl
from jax.experimental.pallas import tpu as pltpu
from jax.experimental.pallas import tpu_sc as plsc

_LANES = 16


@functools.cache
def _make_pool(B, L, D, V):
    info = plsc.get_sparse_core_info()
    nw = info.num_cores * info.num_subcores
    bpw = B // nw
    nchunks = D // _LANES

    mesh = plsc.VectorSubcoreMesh(core_axis_name="c", subcore_axis_name="s")

    @functools.partial(
        pl.kernel,
        out_type=jax.ShapeDtypeStruct((B, D), jnp.float32),
        mesh=mesh,
        compiler_params=pltpu.CompilerParams(use_tc_tiling_on_sc=False),
        scratch_types=[
            pltpu.VMEM((bpw * L,), jnp.int32),
            pltpu.VMEM((2, L, D), jnp.float32),
            pltpu.VMEM((bpw, D), jnp.float32),
            pltpu.SemaphoreType.DMA,
            pltpu.SemaphoreType.DMA,
        ],
    )
    def pool(x_hbm, emb_hbm, out_hbm, idx_v, rows_v, pooled_v, sem0, sem1):
        wid = lax.axis_index("s") * info.num_cores + lax.axis_index("c")
        base = wid * bpw
        pltpu.sync_copy(x_hbm.at[pl.ds(base * L, bpw * L)], idx_v)
        sems = (sem0, sem1)

        def start(b, t):
            pltpu.async_copy(
                emb_hbm.at[idx_v.at[pl.ds(b * L, L)]], rows_v.at[t], sems[t]
            )

        def wait(t):
            pltpu.make_async_copy(
                emb_hbm.at[idx_v.at[pl.ds(0, L)]], rows_v.at[t], sems[t]
            ).wait()

        start(0, 0)
        start(1, 1)
        scale = jnp.float32(1.0 / L)

        def acc_body(t):
            def body(r, acc):
                return tuple(
                    acc[i] + rows_v[t, r, pl.ds(_LANES * i, _LANES)]
                    for i in range(nchunks)
                )
            zero = jnp.zeros((_LANES,), jnp.float32)
            return lax.fori_loop(0, L, body, (zero,) * nchunks, unroll=8)

        def outer(j, carry):
            for t in range(2):
                b = 2 * j + t
                wait(t)
                acc = acc_body(t)

                @pl.when(b + 2 < bpw)
                def _():
                    start(b + 2, t)

                for i in range(nchunks):
                    pooled_v[b, pl.ds(_LANES * i, _LANES)] = acc[i] * scale
            return carry

        lax.fori_loop(0, bpw // 2, outer, 0)
        pltpu.sync_copy(pooled_v, out_hbm.at[pl.ds(base, bpw)])

    return pool


@functools.cache
def _make_mlp(B, D, H, C, blk=512):
    def body(p_ref, w1_ref, b1_ref, w2_ref, b2_ref, o_ref):
        h = jnp.dot(p_ref[...], w1_ref[...], preferred_element_type=jnp.float32)
        h = jnp.maximum(h + b1_ref[...], 0.0)
        o_ref[...] = (
            jnp.dot(h, w2_ref[...], preferred_element_type=jnp.float32)
            + b2_ref[...]
        )

    return pl.pallas_call(
        body,
        grid=(B // blk,),
        in_specs=[
            pl.BlockSpec((blk, D), lambda i: (i, 0)),
            pl.BlockSpec((D, H), lambda i: (0, 0)),
            pl.BlockSpec((1, H), lambda i: (0, 0)),
            pl.BlockSpec((H, C), lambda i: (0, 0)),
            pl.BlockSpec((1, C), lambda i: (0, 0)),
        ],
        out_specs=pl.BlockSpec((blk, C), lambda i: (i, 0)),
        out_shape=jax.ShapeDtypeStruct((B, C), jnp.float32),
    )


def kernel(x, emb_table, W1, b1, W2, b2):
    B, L = x.shape
    V, D = emb_table.shape
    H = W1.shape[1]
    C = W2.shape[1]
    pooled = _make_pool(B, L, D, V)(x.reshape(B * L), emb_table)
    return _make_mlp(B, D, H, C)(
        pooled, W1, b1.reshape(1, H), W2, b2.reshape(1, C)
    )
